# baseline (speedup 1.0000x reference)
import jax
import jax.numpy as jnp
from jax import lax
from jax.experimental import pallas as pl
from jax.experimental.pallas import tpu as pltpu
from jax.experimental.pallas import tpu_sc as plsc

_BATCH = 4096
_HIST = 50
_D = 64
_NC = 2
_NS = 16
_NW = _NC * _NS
_ROWS_PER_W = _BATCH // _NW
_CROWS = 16
_CHUNK = _CROWS * _HIST
_NCHUNK = _ROWS_PER_W // _CROWS
_B_PER_W = _ROWS_PER_W * _HIST


def _emb_body(idx_hbm, table_hbm, out_hbm, idx_v, rows_a, rows_b,
              gsem_a, gsem_b, wsem_a, wsem_b):
  wid = lax.axis_index("s") * _NC + lax.axis_index("c")
  rbase = wid * _ROWS_PER_W
  rows = (rows_a, rows_b)
  gsem = (gsem_a, gsem_b)
  wsem = (wsem_a, wsem_b)

  pltpu.sync_copy(idx_hbm.at[pl.ds(wid * _B_PER_W, _B_PER_W)], idx_v)

  def start_gather(i):
    b = i % 2
    return pltpu.async_copy(
        table_hbm.at[idx_v.at[pl.ds(i * _CHUNK, _CHUNK)]], rows[b], gsem[b])

  def start_writes(i):
    b = i % 2
    return [
        pltpu.async_copy(
            rows[b].at[pl.ds(r * _HIST, _HIST), :],
            out_hbm.at[rbase + i * _CROWS + r], wsem[b])
        for r in range(_CROWS)
    ]

  gathers = [None] * _NCHUNK
  writes = [None] * _NCHUNK
  gathers[0] = start_gather(0)
  for i in range(_NCHUNK):
    gathers[i].wait()
    if i + 1 < _NCHUNK:
      if i >= 1:
        for w in writes[i - 1]:
          w.wait()
      gathers[i + 1] = start_gather(i + 1)
    writes[i] = start_writes(i)
  for w in writes[_NCHUNK - 2] + writes[_NCHUNK - 1]:
    w.wait()


_emb_call = pl.kernel(
    _emb_body,
    out_type=jax.ShapeDtypeStruct((_BATCH, _HIST, _D), jnp.float32),
    mesh=plsc.VectorSubcoreMesh(core_axis_name="c", subcore_axis_name="s"),
    scratch_types=[
        pltpu.VMEM((_B_PER_W,), jnp.int32),
        pltpu.VMEM((_CHUNK, _D), jnp.float32),
        pltpu.VMEM((_CHUNK, _D), jnp.float32),
        pltpu.SemaphoreType.DMA,
        pltpu.SemaphoreType.DMA,
        pltpu.SemaphoreType.DMA,
        pltpu.SemaphoreType.DMA,
    ],
    compiler_params=pltpu.CompilerParams(use_tc_tiling_on_sc=False),
)


@jax.jit
def kernel(input, weights):
  idx = input.reshape(_BATCH * _HIST).astype(jnp.int32)
  return _emb_call(idx, weights)


# speedup vs baseline: 1.0661x; 1.0661x over previous
import jax
import jax.numpy as jnp
from jax import lax
from jax.experimental import pallas as pl
from jax.experimental.pallas import tpu as pltpu
from jax.experimental.pallas import tpu_sc as plsc

_BATCH = 4096
_HIST = 50
_D = 64
_B = _BATCH * _HIST
_NC = 2
_NS = 16
_NW = _NC * _NS
_B_PER_W = _B // _NW
_CHUNK = 800
_NCHUNK = _B_PER_W // _CHUNK


def _emb_body(idx_hbm, table_hbm, out_hbm, idx_v, rows_a, rows_b,
              gsem_a, gsem_b, wsem_a, wsem_b):
  wid = lax.axis_index("s") * _NC + lax.axis_index("c")
  base = wid * _B_PER_W
  rows = (rows_a, rows_b)
  gsem = (gsem_a, gsem_b)
  wsem = (wsem_a, wsem_b)

  pltpu.sync_copy(idx_hbm.at[pl.ds(base, _B_PER_W)], idx_v)

  def start_gather(i):
    b = i % 2
    return pltpu.async_copy(
        table_hbm.at[idx_v.at[pl.ds(i * _CHUNK, _CHUNK)]], rows[b], gsem[b])

  gathers = [None] * _NCHUNK
  writes = [None] * _NCHUNK
  gathers[0] = start_gather(0)
  for i in range(_NCHUNK):
    b = i % 2
    gathers[i].wait()
    if i + 1 < _NCHUNK:
      if i >= 1:
        writes[i - 1].wait()
      gathers[i + 1] = start_gather(i + 1)
    writes[i] = pltpu.async_copy(
        rows[b], out_hbm.at[pl.ds(base + i * _CHUNK, _CHUNK)], wsem[b])
  writes[_NCHUNK - 2].wait()
  writes[_NCHUNK - 1].wait()


_emb_call = pl.kernel(
    _emb_body,
    out_type=jax.ShapeDtypeStruct((_B, _D), jnp.float32),
    mesh=plsc.VectorSubcoreMesh(core_axis_name="c", subcore_axis_name="s"),
    scratch_types=[
        pltpu.VMEM((_B_PER_W,), jnp.int32),
        pltpu.VMEM((_CHUNK, _D), jnp.float32),
        pltpu.VMEM((_CHUNK, _D), jnp.float32),
        pltpu.SemaphoreType.DMA,
        pltpu.SemaphoreType.DMA,
        pltpu.SemaphoreType.DMA,
        pltpu.SemaphoreType.DMA,
    ],
    compiler_params=pltpu.CompilerParams(use_tc_tiling_on_sc=False),
)


@jax.jit
def kernel(input, weights):
  idx_t = input.astype(jnp.int32).T.reshape(_B)
  rows = _emb_call(idx_t, weights)
  return rows.reshape(_HIST, _BATCH, _D).transpose(1, 0, 2)
